# baseline (speedup 1.0000x reference)
import functools

import jax
import jax.numpy as jnp
from jax import lax
from jax.experimental import pallas as pl
from jax.experimental.pallas import tpu as pltpu
from jax.experimental.pallas import tpu_sc as plsc

NC, NS = 2, 16
NW = NC * NS
G = 80
LANES = 16


@functools.lru_cache(maxsize=None)
def _make_collate(B, L, R, C, T, D):
    PER = (B * L) // NW
    BPW = B // NW
    NJ = PER // G
    HALF = NJ // 2
    HROWS = HALF * G

    mesh = plsc.VectorSubcoreMesh(core_axis_name="c", subcore_axis_name="s")
    emb = jax.ShapeDtypeStruct((B * L, D), jnp.float32)

    @functools.partial(
        pl.kernel,
        out_type=[emb] * 9,
        mesh=mesh,
        compiler_params=pltpu.CompilerParams(
            use_tc_tiling_on_sc=False, needs_layout_passes=False),
        scratch_types=[
            pltpu.VMEM((PER,), jnp.int32),
            pltpu.VMEM((PER,), jnp.int32),
            pltpu.VMEM((BPW,), jnp.int32),
            pltpu.VMEM((NJ, G), jnp.int32),
            pltpu.VMEM((NJ, G), jnp.int32),
            pltpu.VMEM((NJ, G), jnp.int32),
            pltpu.VMEM((NJ, G), jnp.int32),
            pltpu.VMEM((NJ, G), jnp.int32),
            pltpu.VMEM((NJ, G), jnp.int32),
            pltpu.VMEM((NJ, G), jnp.int32),
            pltpu.VMEM((HROWS, D), jnp.float32),
            pltpu.VMEM((HROWS, D), jnp.float32),
            pltpu.SemaphoreType.DMA,
            pltpu.SemaphoreType.DMA,
            pltpu.SemaphoreType.DMA,
            pltpu.SemaphoreType.DMA,
        ],
    )
    def collate(road_hbm, cell_hbm, time_hbm, dytab, roadtab, celltab,
                o_dy1, o_dy2, o_dy0, o_ra, o_rb, o_r0, o_c1, o_c2, o_c0,
                road_v, cell_v, time_v,
                i_dy0, i_dy1, i_dy2, i_road, i_cell, i_drev, i_droll,
                buf_a, buf_b, gs_a, gs_b, ws_a, ws_b):
        wid = lax.axis_index("s") * NC + lax.axis_index("c")
        base = wid * PER

        pltpu.sync_copy(road_hbm.at[pl.ds(base, PER)], road_v)
        pltpu.sync_copy(cell_hbm.at[pl.ds(base, PER)], cell_v)
        pltpu.sync_copy(time_hbm.at[pl.ds(wid * BPW, BPW)], time_v)

        iota = lax.iota(jnp.int32, LANES)

        def build(jj, carry):
            for k in range(G // LANES):
                pos = jj * G + k * LANES + iota
                l = pos % L
                t0 = plsc.load_gather(time_v, [pos // L])
                t1 = (t0 + 1) % T
                t2 = (t0 + 2) % T
                s_rev = pos + (L - 1) - 2 * l
                s_roll = pos - l + (l + (L - 1)) % L
                rv = plsc.load_gather(road_v, [pos])
                rrev = plsc.load_gather(road_v, [s_rev])
                rroll = plsc.load_gather(road_v, [s_roll])
                cv = plsc.load_gather(cell_v, [pos])
                sl = pl.ds(k * LANES, LANES)
                i_road[jj, sl] = rv
                i_cell[jj, sl] = cv
                i_dy0[jj, sl] = rv * T + t0
                i_dy1[jj, sl] = rrev * T + t1
                i_dy2[jj, sl] = rroll * T + t2
                i_drev[jj, sl] = base + s_rev
                i_droll[jj, sl] = base + pos - l + (l + 1) % L
            return carry

        lax.fori_loop(0, NJ, build, 0)

        bufs = (buf_a, buf_b)
        gsems = (gs_a, gs_b)
        wsems = (ws_a, ws_b)

        streams = [
            (i_dy0, dytab, ((o_dy0, None),)),
            (i_dy1, dytab, ((o_dy1, None),)),
            (i_dy2, dytab, ((o_dy2, None),)),
            (i_road, roadtab, ((o_r0, None), (o_ra, i_drev), (o_rb, i_droll))),
            (i_cell, celltab, ((o_c0, None), (o_c1, i_drev), (o_c2, i_droll))),
        ]
        tasks = [(idx, tab, writes, h)
                 for idx, tab, writes in streams for h in range(2)]

        def fire_gathers(task, s):
            idx, tab, _writes, h = task
            handles = []
            for j in range(HALF):
                cp = pltpu.make_async_copy(
                    tab.at[idx.at[h * HALF + j]],
                    bufs[s].at[pl.ds(j * G, G)],
                    gsems[s])
                cp.start()
                handles.append(cp)
            return handles

        def fire_writes(task, s):
            _idx, _tab, writes, h = task
            handles = []
            for out, dref in writes:
                if dref is None:
                    cp = pltpu.make_async_copy(
                        bufs[s], out.at[pl.ds(base + h * HROWS, HROWS)],
                        wsems[s])
                    cp.start()
                    handles.append(cp)
                else:
                    for j in range(HALF):
                        cp = pltpu.make_async_copy(
                            bufs[s].at[pl.ds(j * G, G)],
                            out.at[dref.at[h * HALF + j]],
                            wsems[s])
                        cp.start()
                        handles.append(cp)
            return handles

        gh = [None, None]
        wh = [[], []]
        gh[0] = fire_gathers(tasks[0], 0)
        for k, task in enumerate(tasks):
            s = k % 2
            for cp in gh[s]:
                cp.wait()
            if k + 1 < len(tasks):
                for cp in wh[1 - s]:
                    cp.wait()
                wh[1 - s] = []
                gh[1 - s] = fire_gathers(tasks[k + 1], 1 - s)
            wh[s] = fire_writes(task, s)
        for s in (0, 1):
            for cp in wh[s]:
                cp.wait()

    return collate


def kernel(road_idx, cell_idx, time_idx, dytraffic_embs, road_emb2, cell_embs):
    B, L = road_idx.shape
    R, T, D = dytraffic_embs.shape
    C = cell_embs.shape[0]

    road_flat = road_idx.reshape(-1).astype(jnp.int32)
    cell_flat = cell_idx.reshape(-1).astype(jnp.int32)
    tvec = time_idx.astype(jnp.int32)
    dytab = dytraffic_embs.reshape(R * T, D)

    collate = _make_collate(B, L, R, C, T, D)
    dy1, dy2, dy0, ra, rb, r0, c1, c2, c0 = collate(
        road_flat, cell_flat, tvec, dytab, road_emb2, cell_embs)

    sh = (B, L, D)
    lens = jnp.full((B,), L, dtype=jnp.int32)
    t1 = (tvec + 1) % T
    t2 = (tvec + 2) % T
    times = jnp.broadcast_to(tvec[:, None], (B, L))
    times1 = jnp.broadcast_to(t1[:, None], (B, L))
    times2 = jnp.broadcast_to(t2[:, None], (B, L))

    return (dy1.reshape(sh), lens, dy2.reshape(sh), lens, dy0.reshape(sh), lens,
            ra.reshape(sh), lens, rb.reshape(sh), lens, r0.reshape(sh), lens,
            c1.reshape(sh), lens, c2.reshape(sh), lens, c0.reshape(sh), lens,
            times1, times2, times)


# speedup vs baseline: 1.0345x; 1.0345x over previous
import functools

import jax
import jax.numpy as jnp
from jax import lax
from jax.experimental import pallas as pl
from jax.experimental.pallas import tpu as pltpu
from jax.experimental.pallas import tpu_sc as plsc

NC, NS = 2, 16
NW = NC * NS
G = 80
LANES = 16

_SC_PARAMS = pltpu.CompilerParams(
    use_tc_tiling_on_sc=False, needs_layout_passes=False)


def _worker_base(PER):
    wid = lax.axis_index("s") * NC + lax.axis_index("c")
    return wid, wid * PER


def _run_pipeline(tasks, bufs, gsems, wsems, HALF):
    def fire_gathers(task, s):
        handles = []
        for j in range(HALF):
            cp = task[0](j, bufs[s], gsems[s])
            handles.append(cp)
        return handles

    gh = [None, None]
    wh = [[], []]
    gh[0] = fire_gathers(tasks[0], 0)
    for k, task in enumerate(tasks):
        s = k % 2
        for cp in gh[s]:
            cp.wait()
        if k + 1 < len(tasks):
            for cp in wh[1 - s]:
                cp.wait()
            wh[1 - s] = []
            gh[1 - s] = fire_gathers(tasks[k + 1], 1 - s)
        wh[s] = task[1](bufs[s], wsems[s])
    for s in (0, 1):
        for cp in wh[s]:
            cp.wait()


def _start_copy(src, dst, sem):
    cp = pltpu.make_async_copy(src, dst, sem)
    cp.start()
    return cp


@functools.lru_cache(maxsize=None)
def _make_dy_call(B, L, T, D, RT):
    PER = (B * L) // NW
    BPW = B // NW
    NJ = PER // G
    HALF = NJ // 2
    HROWS = HALF * G

    mesh = plsc.VectorSubcoreMesh(core_axis_name="c", subcore_axis_name="s")
    emb = jax.ShapeDtypeStruct((B * L, D), jnp.float32)

    @functools.partial(
        pl.kernel,
        out_type=[emb] * 3,
        mesh=mesh,
        compiler_params=_SC_PARAMS,
        scratch_types=[
            pltpu.VMEM((PER,), jnp.int32),
            pltpu.VMEM((BPW,), jnp.int32),
            pltpu.VMEM((NJ, G), jnp.int32),
            pltpu.VMEM((NJ, G), jnp.int32),
            pltpu.VMEM((NJ, G), jnp.int32),
            pltpu.VMEM((HROWS, D), jnp.float32),
            pltpu.VMEM((HROWS, D), jnp.float32),
            pltpu.SemaphoreType.DMA,
            pltpu.SemaphoreType.DMA,
            pltpu.SemaphoreType.DMA,
            pltpu.SemaphoreType.DMA,
        ],
    )
    def dy_call(road_hbm, time_hbm, dytab,
                o_dy0, o_dy1, o_dy2,
                road_v, time_v, i_dy0, i_dy1, i_dy2,
                buf_a, buf_b, gs_a, gs_b, ws_a, ws_b):
        wid, base = _worker_base(PER)
        pltpu.sync_copy(road_hbm.at[pl.ds(base, PER)], road_v)
        pltpu.sync_copy(time_hbm.at[pl.ds(wid * BPW, BPW)], time_v)

        iota = lax.iota(jnp.int32, LANES)

        def build(jj, carry):
            for k in range(G // LANES):
                pos = jj * G + k * LANES + iota
                l = pos % L
                t0 = plsc.load_gather(time_v, [pos // L])
                s_rev = pos + (L - 1) - 2 * l
                s_roll = pos - l + (l + (L - 1)) % L
                rv = plsc.load_gather(road_v, [pos])
                rrev = plsc.load_gather(road_v, [s_rev])
                rroll = plsc.load_gather(road_v, [s_roll])
                sl = pl.ds(k * LANES, LANES)
                i_dy0[jj, sl] = rv * T + t0
                i_dy1[jj, sl] = rrev * T + (t0 + 1) % T
                i_dy2[jj, sl] = rroll * T + (t0 + 2) % T
            return carry

        lax.fori_loop(0, NJ, build, 0)

        def make_task(idx_ref, out_ref, h):
            def gather(j, buf, sem):
                return _start_copy(dytab.at[idx_ref.at[h * HALF + j]],
                                   buf.at[pl.ds(j * G, G)], sem)

            def writes(buf, sem):
                return [_start_copy(
                    buf, out_ref.at[pl.ds(base + h * HROWS, HROWS)], sem)]

            return (gather, writes)

        tasks = [make_task(i, o, h)
                 for i, o in ((i_dy0, o_dy0), (i_dy1, o_dy1), (i_dy2, o_dy2))
                 for h in range(2)]
        _run_pipeline(tasks, (buf_a, buf_b), (gs_a, gs_b), (ws_a, ws_b), HALF)

    return dy_call


@functools.lru_cache(maxsize=None)
def _make_static_call(B, L, D, N):
    PER = (B * L) // NW
    NJ = PER // G
    HALF = NJ // 2
    HROWS = HALF * G

    mesh = plsc.VectorSubcoreMesh(core_axis_name="c", subcore_axis_name="s")
    emb = jax.ShapeDtypeStruct((B * L, D), jnp.float32)

    @functools.partial(
        pl.kernel,
        out_type=[emb] * 3,
        mesh=mesh,
        compiler_params=_SC_PARAMS,
        scratch_types=[
            pltpu.VMEM((PER,), jnp.int32),
            pltpu.VMEM((NJ, G), jnp.int32),
            pltpu.VMEM((NJ, G), jnp.int32),
            pltpu.VMEM((NJ, G), jnp.int32),
            pltpu.VMEM((HROWS, D), jnp.float32),
            pltpu.VMEM((HROWS, D), jnp.float32),
            pltpu.SemaphoreType.DMA,
            pltpu.SemaphoreType.DMA,
            pltpu.SemaphoreType.DMA,
            pltpu.SemaphoreType.DMA,
        ],
    )
    def static_call(idx_hbm, tab,
                    o_id, o_rev, o_roll,
                    idx_v, i_tab, i_drev, i_droll,
                    buf_a, buf_b, gs_a, gs_b, ws_a, ws_b):
        _wid, base = _worker_base(PER)
        pltpu.sync_copy(idx_hbm.at[pl.ds(base, PER)], idx_v)

        iota = lax.iota(jnp.int32, LANES)

        def build(jj, carry):
            for k in range(G // LANES):
                pos = jj * G + k * LANES + iota
                l = pos % L
                sl = pl.ds(k * LANES, LANES)
                i_tab[jj, sl] = plsc.load_gather(idx_v, [pos])
                i_drev[jj, sl] = base + pos + (L - 1) - 2 * l
                i_droll[jj, sl] = base + pos - l + (l + 1) % L
            return carry

        lax.fori_loop(0, NJ, build, 0)

        def make_task(h):
            def gather(j, buf, sem):
                return _start_copy(tab.at[i_tab.at[h * HALF + j]],
                                   buf.at[pl.ds(j * G, G)], sem)

            def writes(buf, sem):
                handles = [_start_copy(
                    buf, o_id.at[pl.ds(base + h * HROWS, HROWS)], sem)]
                for out_ref, dref in ((o_rev, i_drev), (o_roll, i_droll)):
                    for j in range(HALF):
                        handles.append(_start_copy(
                            buf.at[pl.ds(j * G, G)],
                            out_ref.at[dref.at[h * HALF + j]], sem))
                return handles

            return (gather, writes)

        tasks = [make_task(h) for h in range(2)]
        _run_pipeline(tasks, (buf_a, buf_b), (gs_a, gs_b), (ws_a, ws_b), HALF)

    return static_call


def kernel(road_idx, cell_idx, time_idx, dytraffic_embs, road_emb2, cell_embs):
    B, L = road_idx.shape
    R, T, D = dytraffic_embs.shape
    C = cell_embs.shape[0]

    road_flat = road_idx.reshape(-1).astype(jnp.int32)
    cell_flat = cell_idx.reshape(-1).astype(jnp.int32)
    tvec = time_idx.astype(jnp.int32)
    dytab = dytraffic_embs.reshape(R * T, D)

    dy0, dy1, dy2 = _make_dy_call(B, L, T, D, R * T)(road_flat, tvec, dytab)
    r0, ra, rb = _make_static_call(B, L, D, R)(road_flat, road_emb2)
    c0, c1, c2 = _make_static_call(B, L, D, C)(cell_flat, cell_embs)

    sh = (B, L, D)
    lens = jnp.full((B,), L, dtype=jnp.int32)
    t1 = (tvec + 1) % T
    t2 = (tvec + 2) % T
    times = jnp.broadcast_to(tvec[:, None], (B, L))
    times1 = jnp.broadcast_to(t1[:, None], (B, L))
    times2 = jnp.broadcast_to(t2[:, None], (B, L))

    return (dy1.reshape(sh), lens, dy2.reshape(sh), lens, dy0.reshape(sh), lens,
            ra.reshape(sh), lens, rb.reshape(sh), lens, r0.reshape(sh), lens,
            c1.reshape(sh), lens, c2.reshape(sh), lens, c0.reshape(sh), lens,
            times1, times2, times)
